# baseline (device time: 25431 ns/iter reference)
import jax
import jax.numpy as jnp
from jax import lax
from jax.experimental import pallas as pl
from jax.experimental.pallas import tpu as pltpu

CH = 112
NM = 3
TAIL = 16
K = NM + 1
NS = (NM + 1) // 2


def kernel(x):
    _, m, n = x.shape
    half = n // 2

    def body(x_ref, out_ref, C, xs_s, xs_r, ys_s, ys_r, zs_l, zs_r,
             zr_l, zr_r):
        my_x = lax.axis_index("x")
        my_y = lax.axis_index("y")
        my_z = lax.axis_index("z")
        ox = 1 - my_x
        oy = 1 - my_y
        pe = (my_z % 2) == 0
        y0 = my_y == 0

        X_start = jnp.where(pe, 336, 0)
        Y_start = jnp.where(pe, 680, 672)
        X_tail = jnp.where(pe, 664, 320)
        xs_start = jnp.where(y0, X_start, Y_start)
        xs_tail = jnp.where(y0, X_tail, 1008)
        ys_start = jnp.where(y0, Y_start, X_start)
        ys_tail = jnp.where(y0, 1008, X_tail)

        def xrows(j):
            return (xs_start + CH * j, CH) if j < NM else (xs_tail, TAIL)

        def yrows(j):
            return (ys_start + CH * j, CH) if j < NM else (ys_tail, TAIL)

        barrier = pltpu.get_barrier_semaphore()
        pl.semaphore_signal(barrier, inc=1, device_id=(ox, my_y, my_z),
                            device_id_type=pl.DeviceIdType.MESH)
        pl.semaphore_signal(barrier, inc=1, device_id=(my_x, oy, my_z),
                            device_id_type=pl.DeviceIdType.MESH)

        @pl.when(my_z > 0)
        def _():
            pl.semaphore_signal(barrier, inc=1,
                                device_id=(my_x, my_y, my_z - 1),
                                device_id_type=pl.DeviceIdType.MESH)

        @pl.when(my_z == 0)
        def _():
            pl.semaphore_signal(barrier, inc=1)

        @pl.when(my_z < 3)
        def _():
            pl.semaphore_signal(barrier, inc=1,
                                device_id=(my_x, my_y, my_z + 1),
                                device_id_type=pl.DeviceIdType.MESH)

        @pl.when(my_z == 3)
        def _():
            pl.semaphore_signal(barrier, inc=1)

        pl.semaphore_wait(barrier, 4)

        x_rdmas = []
        for j in range(K):
            rs, sz = xrows(j)
            r = pltpu.make_async_remote_copy(
                src_ref=x_ref.at[0, pl.ds(rs, sz), pl.ds(ox * half, half)],
                dst_ref=C.at[pl.ds(rs, sz), :],
                send_sem=xs_s.at[j], recv_sem=xs_r.at[j],
                device_id=(ox, my_y, my_z),
                device_id_type=pl.DeviceIdType.MESH,
            )
            r.start()
            x_rdmas.append(r)

        def make_y(j):
            rs, sz = xrows(j)
            return pltpu.make_async_remote_copy(
                src_ref=C.at[pl.ds(rs, sz), :],
                dst_ref=C.at[pl.ds(rs, sz), :],
                send_sem=ys_s.at[j], recv_sem=ys_r.at[j],
                device_id=(my_x, oy, my_z),
                device_id_type=pl.DeviceIdType.MESH,
            )

        def make_z_left(c):
            rstart = jnp.where(pe, 336 + CH * c, CH * c)
            return pltpu.make_async_remote_copy(
                src_ref=C.at[pl.ds(rstart, CH), :],
                dst_ref=C.at[pl.ds(rstart, CH), :],
                send_sem=zs_l.at[c], recv_sem=zr_r.at[c],
                device_id=(my_x, my_y, jnp.maximum(my_z - 1, 0)),
                device_id_type=pl.DeviceIdType.MESH,
            )

        def make_z_right(c):
            rstart = jnp.where(pe, 336 + CH * c, CH * c)
            return pltpu.make_async_remote_copy(
                src_ref=C.at[pl.ds(rstart, CH), :],
                dst_ref=C.at[pl.ds(rstart, CH), :],
                send_sem=zs_r.at[c], recv_sem=zr_l.at[c],
                device_id=(my_x, my_y, jnp.minimum(my_z + 1, 3)),
                device_id_type=pl.DeviceIdType.MESH,
            )

        def lcond(c):
            return ((my_z == 1) | (my_z == 2)) if c < NS else \
                ((my_z == 1) | (my_z == 3))

        def rcond(c):
            return ((my_z == 1) | (my_z == 2)) if c < NS else \
                ((my_z == 0) | (my_z == 2))

        def issue_z(c):
            if c >= NM:
                return

            @pl.when(lcond(c))
            def _():
                make_z_left(c).start()

            @pl.when(rcond(c))
            def _():
                make_z_right(c).start()

        def add_rows(rs, sz):
            out_ref[pl.ds(rs, sz), :] = (
                x_ref[0, pl.ds(rs, sz), pl.ds(my_x * half, half)]
                + C[pl.ds(rs, sz), :]
            )

        y_rdmas = []
        for j in range(K):
            x_rdmas[j].wait_recv()
            fwd = make_y(j)
            fwd.start()
            y_rdmas.append(fwd)
            add_rows(*xrows(j))
            if j >= 2:
                y_rdmas[j - 2].wait_recv()
                issue_z(j - 2)
                add_rows(*yrows(j - 2))
        for j in (K - 2, K - 1):
            y_rdmas[j].wait_recv()
            issue_z(j)
            add_rows(*yrows(j))

        for c in range(NM):
            zrs = jnp.where(pe, CH * c, 336 + CH * c)
            wl = ((my_z == 2) | (my_z == 3)) if c < NS else \
                ((my_z == 1) | (my_z == 3))
            wr = ((my_z == 0) | (my_z == 1)) if c < NS else \
                ((my_z == 0) | (my_z == 2))

            @pl.when(wl)
            def _(c=c, zrs=zrs):
                desc = pltpu.make_async_remote_copy(
                    src_ref=C.at[pl.ds(zrs, CH), :],
                    dst_ref=C.at[pl.ds(zrs, CH), :],
                    send_sem=zs_l.at[c], recv_sem=zr_l.at[c],
                    device_id=(my_x, my_y, jnp.maximum(my_z - 1, 0)),
                    device_id_type=pl.DeviceIdType.MESH,
                )
                desc.wait_recv()
                add_rows(zrs, CH)

            @pl.when(wr)
            def _(c=c, zrs=zrs):
                desc = pltpu.make_async_remote_copy(
                    src_ref=C.at[pl.ds(zrs, CH), :],
                    dst_ref=C.at[pl.ds(zrs, CH), :],
                    send_sem=zs_r.at[c], recv_sem=zr_r.at[c],
                    device_id=(my_x, my_y, jnp.minimum(my_z + 1, 3)),
                    device_id_type=pl.DeviceIdType.MESH,
                )
                desc.wait_recv()
                add_rows(zrs, CH)

        for j in range(K):
            x_rdmas[j].wait_send()
            y_rdmas[j].wait_send()
        for c in range(NM):
            @pl.when(lcond(c))
            def _(c=c):
                make_z_left(c).wait_send()

            @pl.when(rcond(c))
            def _(c=c):
                make_z_right(c).wait_send()

    return pl.pallas_call(
        body,
        out_shape=jax.ShapeDtypeStruct((m, half), x.dtype),
        in_specs=[pl.BlockSpec(memory_space=pltpu.VMEM)],
        out_specs=pl.BlockSpec(memory_space=pltpu.VMEM),
        scratch_shapes=[
            pltpu.VMEM((m, half), x.dtype),
            pltpu.SemaphoreType.DMA((K,)),
            pltpu.SemaphoreType.DMA((K,)),
            pltpu.SemaphoreType.DMA((K,)),
            pltpu.SemaphoreType.DMA((K,)),
            pltpu.SemaphoreType.DMA((NM,)),
            pltpu.SemaphoreType.DMA((NM,)),
            pltpu.SemaphoreType.DMA((NM,)),
            pltpu.SemaphoreType.DMA((NM,)),
        ],
        compiler_params=pltpu.CompilerParams(collective_id=0),
    )(x)


# device time: 20539 ns/iter; 1.2382x vs baseline; 1.2382x over previous
import jax
import jax.numpy as jnp
from jax import lax
from jax.experimental import pallas as pl
from jax.experimental.pallas import tpu as pltpu

CH = 48
NM = 7
TAIL = 16
K = NM + 1
NS = (NM + 1) // 2


def kernel(x):
    _, m, n = x.shape
    half = n // 2

    def body(x_ref, out_ref, C, xs_s, xs_r, ys_s, ys_r, zs_l, zs_r,
             zr_l, zr_r):
        my_x = lax.axis_index("x")
        my_y = lax.axis_index("y")
        my_z = lax.axis_index("z")
        ox = 1 - my_x
        oy = 1 - my_y
        pe = (my_z % 2) == 0
        y0 = my_y == 0

        X_start = jnp.where(pe, 336, 0)
        Y_start = jnp.where(pe, 680, 672)
        X_tail = jnp.where(pe, 664, 320)
        xs_start = jnp.where(y0, X_start, Y_start)
        xs_tail = jnp.where(y0, X_tail, 1008)
        ys_start = jnp.where(y0, Y_start, X_start)
        ys_tail = jnp.where(y0, 1008, X_tail)

        def xrows(j):
            return (xs_start + CH * j, CH) if j < NM else (xs_tail, TAIL)

        def yrows(j):
            return (ys_start + CH * j, CH) if j < NM else (ys_tail, TAIL)

        barrier = pltpu.get_barrier_semaphore()
        pl.semaphore_signal(barrier, inc=1, device_id=(ox, my_y, my_z),
                            device_id_type=pl.DeviceIdType.MESH)
        pl.semaphore_signal(barrier, inc=1, device_id=(my_x, oy, my_z),
                            device_id_type=pl.DeviceIdType.MESH)

        @pl.when(my_z > 0)
        def _():
            pl.semaphore_signal(barrier, inc=1,
                                device_id=(my_x, my_y, my_z - 1),
                                device_id_type=pl.DeviceIdType.MESH)

        @pl.when(my_z == 0)
        def _():
            pl.semaphore_signal(barrier, inc=1)

        @pl.when(my_z < 3)
        def _():
            pl.semaphore_signal(barrier, inc=1,
                                device_id=(my_x, my_y, my_z + 1),
                                device_id_type=pl.DeviceIdType.MESH)

        @pl.when(my_z == 3)
        def _():
            pl.semaphore_signal(barrier, inc=1)

        pl.semaphore_wait(barrier, 4)

        x_rdmas = []
        for j in range(K):
            rs, sz = xrows(j)
            r = pltpu.make_async_remote_copy(
                src_ref=x_ref.at[0, pl.ds(rs, sz), pl.ds(ox * half, half)],
                dst_ref=C.at[pl.ds(rs, sz), :],
                send_sem=xs_s.at[j], recv_sem=xs_r.at[j],
                device_id=(ox, my_y, my_z),
                device_id_type=pl.DeviceIdType.MESH,
            )
            r.start()
            x_rdmas.append(r)

        def make_y(j):
            rs, sz = xrows(j)
            return pltpu.make_async_remote_copy(
                src_ref=C.at[pl.ds(rs, sz), :],
                dst_ref=C.at[pl.ds(rs, sz), :],
                send_sem=ys_s.at[j], recv_sem=ys_r.at[j],
                device_id=(my_x, oy, my_z),
                device_id_type=pl.DeviceIdType.MESH,
            )

        def make_z_left(c):
            rstart = jnp.where(pe, 336 + CH * c, CH * c)
            return pltpu.make_async_remote_copy(
                src_ref=C.at[pl.ds(rstart, CH), :],
                dst_ref=C.at[pl.ds(rstart, CH), :],
                send_sem=zs_l.at[c], recv_sem=zr_r.at[c],
                device_id=(my_x, my_y, jnp.maximum(my_z - 1, 0)),
                device_id_type=pl.DeviceIdType.MESH,
            )

        def make_z_right(c):
            rstart = jnp.where(pe, 336 + CH * c, CH * c)
            return pltpu.make_async_remote_copy(
                src_ref=C.at[pl.ds(rstart, CH), :],
                dst_ref=C.at[pl.ds(rstart, CH), :],
                send_sem=zs_r.at[c], recv_sem=zr_l.at[c],
                device_id=(my_x, my_y, jnp.minimum(my_z + 1, 3)),
                device_id_type=pl.DeviceIdType.MESH,
            )

        def lcond(c):
            return ((my_z == 1) | (my_z == 2)) if c < NS else \
                ((my_z == 1) | (my_z == 3))

        def rcond(c):
            return ((my_z == 1) | (my_z == 2)) if c < NS else \
                ((my_z == 0) | (my_z == 2))

        def issue_z(c, pred):
            if c >= NM:
                return

            @pl.when(lcond(c) & pred)
            def _():
                make_z_left(c).start()

            @pl.when(rcond(c) & pred)
            def _():
                make_z_right(c).start()

        def add_rows(rs, sz):
            out_ref[pl.ds(rs, sz), :] = (
                x_ref[0, pl.ds(rs, sz), pl.ds(my_x * half, half)]
                + C[pl.ds(rs, sz), :]
            )

        y_rdmas = []
        for j in range(K):
            x_rdmas[j].wait_recv()
            fwd = make_y(j)
            fwd.start()
            y_rdmas.append(fwd)
            add_rows(*xrows(j))
            issue_z(j, y0)
            if j >= 2:
                y_rdmas[j - 2].wait_recv()
                issue_z(j - 2, jnp.logical_not(y0))
                add_rows(*yrows(j - 2))
        for j in (K - 2, K - 1):
            y_rdmas[j].wait_recv()
            issue_z(j, jnp.logical_not(y0))
            add_rows(*yrows(j))

        for c in range(NM):
            zrs = jnp.where(pe, CH * c, 336 + CH * c)
            wl = ((my_z == 2) | (my_z == 3)) if c < NS else \
                ((my_z == 1) | (my_z == 3))
            wr = ((my_z == 0) | (my_z == 1)) if c < NS else \
                ((my_z == 0) | (my_z == 2))

            @pl.when(wl)
            def _(c=c, zrs=zrs):
                desc = pltpu.make_async_remote_copy(
                    src_ref=C.at[pl.ds(zrs, CH), :],
                    dst_ref=C.at[pl.ds(zrs, CH), :],
                    send_sem=zs_l.at[c], recv_sem=zr_l.at[c],
                    device_id=(my_x, my_y, jnp.maximum(my_z - 1, 0)),
                    device_id_type=pl.DeviceIdType.MESH,
                )
                desc.wait_recv()
                add_rows(zrs, CH)

            @pl.when(wr)
            def _(c=c, zrs=zrs):
                desc = pltpu.make_async_remote_copy(
                    src_ref=C.at[pl.ds(zrs, CH), :],
                    dst_ref=C.at[pl.ds(zrs, CH), :],
                    send_sem=zs_r.at[c], recv_sem=zr_r.at[c],
                    device_id=(my_x, my_y, jnp.minimum(my_z + 1, 3)),
                    device_id_type=pl.DeviceIdType.MESH,
                )
                desc.wait_recv()
                add_rows(zrs, CH)

        for j in range(K):
            x_rdmas[j].wait_send()
            y_rdmas[j].wait_send()
        for c in range(NM):
            @pl.when(lcond(c))
            def _(c=c):
                make_z_left(c).wait_send()

            @pl.when(rcond(c))
            def _(c=c):
                make_z_right(c).wait_send()

    return pl.pallas_call(
        body,
        out_shape=jax.ShapeDtypeStruct((m, half), x.dtype),
        in_specs=[pl.BlockSpec(memory_space=pltpu.VMEM)],
        out_specs=pl.BlockSpec(memory_space=pltpu.VMEM),
        scratch_shapes=[
            pltpu.VMEM((m, half), x.dtype),
            pltpu.SemaphoreType.DMA((K,)),
            pltpu.SemaphoreType.DMA((K,)),
            pltpu.SemaphoreType.DMA((K,)),
            pltpu.SemaphoreType.DMA((K,)),
            pltpu.SemaphoreType.DMA((NM,)),
            pltpu.SemaphoreType.DMA((NM,)),
            pltpu.SemaphoreType.DMA((NM,)),
            pltpu.SemaphoreType.DMA((NM,)),
        ],
        compiler_params=pltpu.CompilerParams(collective_id=0),
    )(x)
